# baseline (speedup 1.0000x reference)
import functools

import jax
import jax.numpy as jnp
import numpy as np
from jax import lax
from jax.experimental import pallas as pl
from jax.experimental.pallas import tpu as pltpu
from jax.experimental.pallas import tpu_sc as plsc

_B, _N, _G = 8, 20000, 64
_POS_THR, _NEG_THR = 0.5, 0.1
_SAMPLES = 512
_POS_QUOTA = 128
_L1W = 1.0

_BLK = 2048
_NP = 20480


def _iou_body(pr_ref, gt_ref, mx_ref, am_ref):
    pr = pr_ref[0]
    gt = gt_ref[0]
    px1 = pr[0:1, :]
    py1 = pr[1:2, :]
    px2 = pr[2:3, :]
    py2 = pr[3:4, :]
    gx1 = gt[:, 0:1]
    gy1 = gt[:, 1:2]
    gx2 = gt[:, 2:3]
    gy2 = gt[:, 3:4]
    x1 = jnp.maximum(px1, gx1)
    y1 = jnp.maximum(py1, gy1)
    x2 = jnp.minimum(px2, gx2)
    y2 = jnp.minimum(py2, gy2)
    inter = jnp.maximum(x2 - x1, 0.0) * jnp.maximum(y2 - y1, 0.0)
    ap = (px2 - px1) * (py2 - py1)
    ag = (gx2 - gx1) * (gy2 - gy1)
    union = ap + ag - inter
    iou = inter / jnp.maximum(union, 1e-8)
    mx = jnp.max(iou, axis=0, keepdims=True)
    gidx = lax.broadcasted_iota(jnp.int32, (_G, 1), 0)
    am = jnp.min(jnp.where(iou == mx, gidx, _G), axis=0, keepdims=True)
    mx_ref[...] = mx[None]
    am_ref[...] = am[None]


def _max_argmax(proposals, gt_boxes):
    pr_t = jnp.transpose(proposals, (0, 2, 1))
    pr_t = jnp.pad(pr_t, ((0, 0), (0, 0), (0, _NP - _N)))
    grid = (_B, _NP // _BLK)
    mx, am = pl.pallas_call(
        _iou_body,
        grid=grid,
        in_specs=[
            pl.BlockSpec((1, 4, _BLK), lambda b, n: (b, 0, n)),
            pl.BlockSpec((1, _G, 4), lambda b, n: (b, 0, 0)),
        ],
        out_specs=[
            pl.BlockSpec((1, 1, _BLK), lambda b, n: (b * (_NP // _BLK) + n, 0, 0)),
            pl.BlockSpec((1, 1, _BLK), lambda b, n: (b * (_NP // _BLK) + n, 0, 0)),
        ],
        out_shape=[
            jax.ShapeDtypeStruct((_B * (_NP // _BLK), 1, _BLK), jnp.float32),
            jax.ShapeDtypeStruct((_B * (_NP // _BLK), 1, _BLK), jnp.int32),
        ],
    )(pr_t, gt_boxes)
    mx = mx.reshape(_B, _NP)
    am = am.reshape(_B, _NP)
    return mx[:, :_N], am[:, :_N]


def _centrehw(b):
    w = b[..., 2] - b[..., 0]
    h = b[..., 3] - b[..., 1]
    return jnp.stack([b[..., 0] + 0.5 * w, b[..., 1] + 0.5 * h, w, h], axis=-1)


_NV = _NP // 16
_PCAP = 2048
_NCAP = 3072
_ZCAP = 544
_NBINS = 512
_FILLW = 1280


def _splat(x):
    return jnp.broadcast_to(x, (16,))


def _prefix16(x_i32, lane, hs_v):
    y = x_i32
    for k in (1, 2, 4, 8):
        hs_v[pl.ds(0, 16)] = y
        g = plsc.load_gather(hs_v, [jnp.maximum(lane - k, 0)])
        y = y + g * (lane >= k).astype(jnp.int32)
    return y


def _sc_body(iou_hbm, keep_hbm, npos_hbm,
             iou_v, pv_v, pi_v, zb_v, nv_v, ni_v,
             hist_v, keep_v, flags_v, fill_v, hs_v, rk_v, npos_v):
    wid = lax.axis_index("s") * 2 + lax.axis_index("c")

    @pl.when(wid < _B)
    def _():
        b = wid
        pltpu.sync_copy(iou_hbm.at[pl.ds(b * _NP, _NP)], iou_v)

        lane = lax.iota(jnp.int32, 16)
        zero16i = jnp.zeros((16,), jnp.int32)
        one16i = jnp.full((16,), 1, jnp.int32)
        lane512 = lane * 512
        pcap16 = jnp.full((16,), _PCAP, jnp.int32)
        zcap16 = jnp.full((16,), _ZCAP, jnp.int32)
        ncap16 = jnp.full((16,), _NCAP, jnp.int32)

        def _init(i, _):
            hist_v[pl.ds(i * 16, 16)] = zero16i
            return 0
        lax.fori_loop(0, _NBINS * 16 // 16, _init, 0)

        def _initf(i, _):
            flags_v[pl.ds(i * 16, 16)] = zero16i
            return 0
        lax.fori_loop(0, _FILLW // 16, _initf, 0)

        sent_p = jnp.full((16,), -1.0, jnp.float32)
        def _initp(i, _):
            pv_v[pl.ds(i * 16, 16)] = sent_p
            return 0
        lax.fori_loop(0, _PCAP // 16, _initp, 0)

        sent_n = jnp.full((16,), 1e9, jnp.float32)
        def _initn(i, _):
            nv_v[pl.ds(i * 16, 16)] = sent_n
            return 0
        lax.fori_loop(0, _NCAP // 16, _initn, 0)

        def _scan1(i, carry):
            poff, zoff = carry
            v = iou_v[pl.ds(i * 16, 16)]
            idxv = lane + i * 16
            pos = v >= _POS_THR
            neg = v < _NEG_THR
            zero = v == 0.0
            negnz = neg & jnp.logical_not(zero)
            pc = _prefix16(pos.astype(jnp.int32), lane, hs_v)
            pslot = (_splat(poff) - 1) + pc
            plsc.store_scatter(pv_v, [pslot], v, mask=pos & (pslot < pcap16))
            plsc.store_scatter(pi_v, [pslot], idxv,
                               mask=pos & (pslot < pcap16))
            zc = _prefix16(zero.astype(jnp.int32), lane, hs_v)
            zslot = (_splat(zoff) - 1) + zc
            plsc.store_scatter(zb_v, [zslot], idxv,
                               mask=zero & (zslot < zcap16))
            binv = jnp.minimum((v * 5120.0).astype(jnp.int32), 511)
            plsc.addupdate_scatter(hist_v, [lane512 + binv], one16i,
                                   mask=negnz)
            return poff + jnp.max(pc), zoff + jnp.max(zc)

        cnt_pos, cnt_zero = lax.fori_loop(
            0, _NV, _scan1, (jnp.int32(0), jnp.int32(0)))

        hs_v[pl.ds(0, 16)] = zero16i

        def _htot(i, _):
            s = hs_v[pl.ds(0, 16)]
            for l in range(16):
                s = s + hist_v[pl.ds(l * 512 + i * 16, 16)]
            hs_v[pl.ds(0, 16)] = s
            return 0
        lax.fori_loop(0, _NBINS // 16, _htot, 0)
        cnt_nz = jnp.sum(hs_v[pl.ds(0, 16)])

        cnt_neg = cnt_zero + cnt_nz
        n_pos = jnp.minimum(cnt_pos, _POS_QUOTA)
        t_neg = jnp.minimum(cnt_neg, _SAMPLES - n_pos)
        zt = jnp.minimum(t_neg, cnt_zero)
        t2 = t_neg - zt

        npos16 = _splat(n_pos)

        @pl.when(t2 > 0)
        def _():
            t2v = _splat(t2)

            def _hscan(i, carry):
                base, c_s = carry
                s = zero16i
                for l in range(16):
                    s = s + hist_v[pl.ds(l * 512 + i * 16, 16)]
                cum = _prefix16(s, lane, hs_v) + _splat(base)
                hit = cum >= t2v
                nh = jnp.max(plsc.all_reduce_population_count(hit))
                c_here = i * 16 + (16 - nh)
                first = (c_s < 0) & (nh > 0)
                c_new = jnp.where(first, c_here, c_s)
                return jnp.max(cum), c_new

            _, c_s = lax.fori_loop(
                0, _NBINS // 16, _hscan, (jnp.int32(0), jnp.int32(-1)))
            cutoff = _splat(c_s)

            def _scan2(i, noff):
                v = iou_v[pl.ds(i * 16, 16)]
                idxv = lane + i * 16
                neg = v < _NEG_THR
                zero = v == 0.0
                negnz = neg & jnp.logical_not(zero)
                binv = jnp.minimum((v * 5120.0).astype(jnp.int32), 511)
                cand = negnz & (binv <= cutoff)
                cc = _prefix16(cand.astype(jnp.int32), lane, hs_v)
                cslot = (_splat(noff) - 1) + cc
                plsc.store_scatter(nv_v, [cslot], v,
                                   mask=cand & (cslot < ncap16))
                plsc.store_scatter(ni_v, [cslot], idxv,
                                   mask=cand & (cslot < ncap16))
                return noff + jnp.max(cc)
            noff = lax.fori_loop(0, _NV, _scan2, jnp.int32(0))

            nn_c = jnp.minimum(noff, _NCAP)
            nvecs = lax.shift_right_logical(nn_c + 15, 4)
            zt16 = _splat(zt)
            t216 = _splat(t2)
            nn16 = _splat(nn_c)

            def _rank_n(i, _):
                vi = nv_v[pl.ds(i * 16, 16)]
                xi = ni_v[pl.ds(i * 16, 16)]
                rk_v[pl.ds(0, 16)] = zero16i
                def _inner(j, _):
                    rank = rk_v[pl.ds(0, 16)]
                    for r in range(16):
                        rot = (lane + r) & 15
                        vr = plsc.load_gather(nv_v, [j * 16 + rot])
                        xr = plsc.load_gather(ni_v, [j * 16 + rot])
                        less = (vr < vi) | ((vr == vi) & (xr < xi))
                        rank = rank + less.astype(jnp.int32)
                    rk_v[pl.ds(0, 16)] = rank
                    return 0
                lax.fori_loop(0, nvecs, _inner, 0)
                rank = rk_v[pl.ds(0, 16)]
                sel = ((lane + i * 16) < nn16) & (rank < t216)
                plsc.store_scatter(keep_v, [npos16 + zt16 + rank], xi,
                                   mask=sel)
                return 0
            lax.fori_loop(0, nvecs, _rank_n, 0)

        np_c = jnp.minimum(cnt_pos, _PCAP)
        pvecs = lax.shift_right_logical(np_c + 15, 4)
        npc16 = _splat(np_c)

        def _rank_p(i, _):
            vi = pv_v[pl.ds(i * 16, 16)]
            xi = pi_v[pl.ds(i * 16, 16)]
            rk_v[pl.ds(0, 16)] = zero16i
            def _inner(j, _):
                rank = rk_v[pl.ds(0, 16)]
                for r in range(16):
                    rot = (lane + r) & 15
                    vr = plsc.load_gather(pv_v, [j * 16 + rot])
                    xr = plsc.load_gather(pi_v, [j * 16 + rot])
                    before = (vr > vi) | ((vr == vi) & (xr < xi))
                    rank = rank + before.astype(jnp.int32)
                rk_v[pl.ds(0, 16)] = rank
                return 0
            lax.fori_loop(0, pvecs, _inner, 0)
            rank = rk_v[pl.ds(0, 16)]
            sel = ((lane + i * 16) < npc16) & (rank < npos16)
            plsc.store_scatter(keep_v, [rank], xi, mask=sel)
            plsc.store_scatter(flags_v, [jnp.minimum(xi, _FILLW - 1)],
                               one16i, mask=sel & (xi < _FILLW))
            return 0
        lax.fori_loop(0, pvecs, _rank_p, 0)

        zt16b = _splat(zt)
        zvecs = lax.shift_right_logical(zt + 15, 4)

        def _zcopy(i, _):
            zi = zb_v[pl.ds(i * 16, 16)]
            k = lane + i * 16
            plsc.store_scatter(keep_v, [npos16 + k], zi, mask=k < zt16b)
            return 0
        lax.fori_loop(0, zvecs, _zcopy, 0)

        n_fill = _SAMPLES - n_pos - t_neg

        @pl.when(n_fill > 0)
        def _():
            def _fcompact(i, foff):
                v = iou_v[pl.ds(i * 16, 16)]
                idxv = lane + i * 16
                neg = v < _NEG_THR
                flg = flags_v[pl.ds(i * 16, 16)]
                unsel = jnp.logical_not(neg) & (flg == 0)
                fc = _prefix16(unsel.astype(jnp.int32), lane, hs_v)
                fslot = (_splat(foff) - 1) + fc
                plsc.store_scatter(fill_v, [fslot], idxv,
                                   mask=unsel & (fslot < zcap16))
                return foff + jnp.max(fc)
            lax.fori_loop(0, _FILLW // 16, _fcompact, jnp.int32(0))

            base16 = _splat(n_pos + t_neg)
            nf16 = _splat(n_fill)

            def _fcopy(i, _):
                fi = fill_v[pl.ds(i * 16, 16)]
                k = lane + i * 16
                plsc.store_scatter(keep_v, [base16 + k], fi, mask=k < nf16)
                return 0
            lax.fori_loop(0, _SAMPLES // 16, _fcopy, 0)

        npos_v[pl.ds(0, 16)] = npos16
        pltpu.sync_copy(keep_v, keep_hbm.at[pl.ds(b * _SAMPLES, _SAMPLES)])
        pltpu.sync_copy(npos_v, npos_hbm.at[pl.ds(b * 16, 16)])


def _sc_select(iou):
    mesh = plsc.VectorSubcoreMesh(core_axis_name="c", subcore_axis_name="s")
    f = pl.kernel(
        _sc_body,
        out_type=[
            jax.ShapeDtypeStruct((_B * _SAMPLES,), jnp.int32),
            jax.ShapeDtypeStruct((_B * 16,), jnp.int32),
        ],
        mesh=mesh,
        compiler_params=pltpu.CompilerParams(needs_layout_passes=False),
        scratch_types=[
            pltpu.VMEM((_NP,), jnp.float32),
            pltpu.VMEM((_PCAP,), jnp.float32),
            pltpu.VMEM((_PCAP,), jnp.int32),
            pltpu.VMEM((_ZCAP,), jnp.int32),
            pltpu.VMEM((_NCAP,), jnp.float32),
            pltpu.VMEM((_NCAP,), jnp.int32),
            pltpu.VMEM((_NBINS * 16,), jnp.int32),
            pltpu.VMEM((_SAMPLES,), jnp.int32),
            pltpu.VMEM((_FILLW,), jnp.int32),
            pltpu.VMEM((_ZCAP,), jnp.int32),
            pltpu.VMEM((16,), jnp.int32),
            pltpu.VMEM((16,), jnp.int32),
            pltpu.VMEM((16,), jnp.int32),
        ],
    )
    keep, npos = f(iou.reshape(-1))
    return keep.reshape(_B, _SAMPLES), npos.reshape(_B, 16)


def kernel(proposals, gt_boxes):
    max_iou, argmax_gt = _max_argmax(proposals, gt_boxes)
    _sc_keep, _sc_npos = _sc_select(max_iou)

    idx_dtype = jnp.int32
    pos_mask = max_iou >= _POS_THR
    neg_mask = max_iou < _NEG_THR
    n_pos = jnp.minimum(jnp.sum(pos_mask, axis=1), _POS_QUOTA)[:, None]
    n_neg = jnp.minimum(jnp.sum(neg_mask, axis=1)[:, None], _SAMPLES - n_pos)

    _, pos_idx = lax.top_k(jnp.where(pos_mask, max_iou, -1.0), _POS_QUOTA)
    _, neg_idx = lax.top_k(jnp.where(neg_mask, -max_iou, -2.0), _SAMPLES)

    _FILLW = 1280
    bsz = proposals.shape[0]
    b_idx = jnp.arange(bsz)[:, None]
    s_idx = jnp.arange(_SAMPLES)[None, :]
    unsel = jnp.ones((bsz, _FILLW), jnp.bool_)
    pos_in = (s_idx[:, :_POS_QUOTA] < n_pos) & (pos_idx < _FILLW)
    unsel = unsel.at[b_idx, jnp.where(pos_in, pos_idx, _FILLW)].set(
        False, mode='drop')
    neg_in = (s_idx < n_neg) & (neg_idx < _FILLW)
    unsel = unsel.at[b_idx, jnp.where(neg_in, neg_idx, _FILLW)].set(
        False, mode='drop')
    fill_key = jnp.where(unsel, -jnp.arange(_FILLW, dtype=idx_dtype)[None, :],
                         -_FILLW - 1)
    _, fill_idx = lax.top_k(fill_key, _SAMPLES)

    take = functools.partial(jnp.take_along_axis, axis=1)
    sn = jnp.clip(s_idx - n_pos, 0, _SAMPLES - 1)
    sf = jnp.clip(s_idx - n_pos - n_neg, 0, _SAMPLES - 1)
    keep_idx = jnp.where(
        s_idx < n_pos, take(pos_idx, jnp.clip(s_idx, 0, _POS_QUOTA - 1)),
        jnp.where(s_idx < n_pos + n_neg, take(neg_idx, sn),
                  take(fill_idx, sf))).astype(idx_dtype)
    batch_labels = (s_idx < _sc_npos[:, :1]).astype(idx_dtype)
    ag = argmax_gt[b_idx, keep_idx]

    roi_batch = proposals[b_idx, keep_idx]
    gt_batch = gt_boxes[b_idx, ag]
    gt_c = _centrehw(gt_batch)
    roi_c = _centrehw(roi_batch)
    dx = (gt_c[..., 0] - roi_c[..., 0]) / roi_c[..., 2]
    dy = (gt_c[..., 1] - roi_c[..., 1]) / roi_c[..., 3]
    dw = jnp.log(gt_c[..., 2] / roi_c[..., 2])
    dh = jnp.log(gt_c[..., 3] / roi_c[..., 3])
    bbox_targets = jnp.stack([dx, dy, dw, dh], axis=-1)
    in_weights = jnp.where((batch_labels == 1)[..., None], _L1W, 0.0) * \
        jnp.ones((1, 1, 4), jnp.float32)
    out_weights = (in_weights > 0).astype(jnp.float32)
    return (roi_batch, batch_labels, bbox_targets, in_weights, out_weights)
